# baseline (speedup 1.0000x reference)
import functools

import jax
import jax.numpy as jnp
from jax import lax
from jax.experimental import pallas as pl
from jax.experimental.pallas import tpu as pltpu
from jax.experimental.pallas import tpu_sc as plsc


_CHUNK_BOUNDS = (0, 2736, 5472, 8192)


def _dist_argmin_body(zn_ref, cn_ref, z_ref, cbt_ref,
                      idx_ref, dmin_ref,
                      v0_ref, i0_ref, v1_ref, i1_ref, v2_ref, i2_ref, *, bk):
    j = pl.program_id(1)
    nk = pl.num_programs(1)
    inf = jnp.float32(jnp.inf)

    @pl.when(j == 0)
    def _():
        for vr, ir in ((v0_ref, i0_ref), (v1_ref, i1_ref), (v2_ref, i2_ref)):
            vr[...] = jnp.full(vr.shape, inf, jnp.float32)
            ir[...] = jnp.zeros(ir.shape, jnp.int32)

    mm = jnp.dot(z_ref[...], cbt_ref[...], preferred_element_type=jnp.float32)
    d = (zn_ref[...] + cn_ref[...]) - 2.0 * mm
    kglob = lax.broadcasted_iota(jnp.int32, d.shape, 1) + j * bk

    def merge(vr, ir, lo, hi):
        msk = (kglob >= lo) & (kglob < hi)
        dm = jnp.where(msk, d, inf)
        lmin = jnp.min(dm, axis=1, keepdims=True)
        larg = jnp.min(jnp.where(dm == lmin, kglob, _CHUNK_BOUNDS[-1]),
                       axis=1, keepdims=True)
        take = lmin < vr[...]
        ir[...] = jnp.where(take, larg, ir[...])
        vr[...] = jnp.where(take, lmin, vr[...])

    refs = ((v0_ref, i0_ref), (v1_ref, i1_ref), (v2_ref, i2_ref))
    for c in range(3):
        lo, hi = _CHUNK_BOUNDS[c], _CHUNK_BOUNDS[c + 1]
        jlo, jhi = lo // bk, (hi - 1) // bk

        @pl.when((j >= jlo) & (j <= jhi))
        def _(c=c, lo=lo, hi=hi):
            merge(refs[c][0], refs[c][1], lo, hi)

    @pl.when(j == nk - 1)
    def _():
        accv = v0_ref[...].astype(jnp.bfloat16).astype(jnp.float32)
        acci = i0_ref[...]
        accraw = v0_ref[...]
        for vr, ir in ((v1_ref, i1_ref), (v2_ref, i2_ref)):
            take = vr[...] < accv
            acci = jnp.where(take, ir[...], acci)
            accraw = jnp.where(take, vr[...], accraw)
            accv = jnp.where(take, vr[...], accv)
            accv = accv.astype(jnp.bfloat16).astype(jnp.float32)
        idx_ref[...] = acci
        dmin_ref[...] = accraw


def _dist_argmin(flat_z, cbt, zn, cn, bm, bk):
    t, d_ = flat_z.shape
    k = cbt.shape[1]
    grid = (t // bm, k // bk)
    return pl.pallas_call(
        functools.partial(_dist_argmin_body, bk=bk),
        grid=grid,
        in_specs=[
            pl.BlockSpec((bm, 1), lambda i, j: (i, 0)),
            pl.BlockSpec((1, bk), lambda i, j: (0, j)),
            pl.BlockSpec((bm, d_), lambda i, j: (i, 0)),
            pl.BlockSpec((d_, bk), lambda i, j: (0, j)),
        ],
        out_specs=[
            pl.BlockSpec((bm, 1), lambda i, j: (i, 0)),
            pl.BlockSpec((bm, 1), lambda i, j: (i, 0)),
        ],
        out_shape=[
            jax.ShapeDtypeStruct((t, 1), jnp.int32),
            jax.ShapeDtypeStruct((t, 1), jnp.float32),
        ],
        scratch_shapes=[
            pltpu.VMEM((bm, 1), jnp.float32),
            pltpu.VMEM((bm, 1), jnp.int32),
            pltpu.VMEM((bm, 1), jnp.float32),
            pltpu.VMEM((bm, 1), jnp.int32),
            pltpu.VMEM((bm, 1), jnp.float32),
            pltpu.VMEM((bm, 1), jnp.int32),
        ],
    )(zn, cn, flat_z, cbt)


def _make_sc_gather(k, d_, t):
    info = plsc.get_sparse_core_info()
    nw = info.num_cores * info.num_subcores
    b_per_w = t // nw
    ch = 128
    n_ch = b_per_w // ch
    mesh = plsc.VectorSubcoreMesh(core_axis_name="c", subcore_axis_name="s")

    @functools.partial(
        pl.kernel, mesh=mesh,
        out_type=jax.ShapeDtypeStruct((t, d_), jnp.float32),
        scratch_types=[
            pltpu.VMEM((b_per_w,), jnp.int32),
            pltpu.VMEM((ch, d_), jnp.float32),
            pltpu.VMEM((ch, d_), jnp.float32),
            pltpu.SemaphoreType.DMA,
            pltpu.SemaphoreType.DMA,
        ],
    )
    def gather_kernel(table_hbm, idx_hbm, out_hbm, idx_v, rows_a, rows_b, sem_a, sem_b):
        wid = lax.axis_index("s") * info.num_cores + lax.axis_index("c")
        base = wid * b_per_w
        pltpu.sync_copy(idx_hbm.at[pl.ds(base, b_per_w)], idx_v)
        bufs = ((rows_a, sem_a), (rows_b, sem_b))
        copies = [None, None]
        for c in range(n_ch):
            rows, sem = bufs[c % 2]
            copies[c % 2] = pltpu.async_copy(
                table_hbm.at[idx_v.at[pl.ds(c * ch, ch)]], rows, sem)
            if c >= 1:
                prows, _ = bufs[(c - 1) % 2]
                copies[(c - 1) % 2].wait()
                pltpu.sync_copy(prows, out_hbm.at[pl.ds(base + (c - 1) * ch, ch)])
        lrows, _ = bufs[(n_ch - 1) % 2]
        copies[(n_ch - 1) % 2].wait()
        pltpu.sync_copy(lrows, out_hbm.at[pl.ds(base + (n_ch - 1) * ch, ch)])

    return gather_kernel


def kernel(z_e, codebook):
    b, n, d_ = z_e.shape
    k = codebook.shape[0]
    t = b * n
    flat_z = z_e.reshape(-1, d_)
    zn = jnp.sum(flat_z ** 2, axis=1, keepdims=True)
    cn = jnp.sum(codebook ** 2, axis=1)[None, :]
    cbt = codebook.T

    idx2d, dmin2d = _dist_argmin(flat_z, cbt, zn, cn, bm=1024, bk=512)
    indices = idx2d.reshape(t)

    z_q_flat = _make_sc_gather(k, d_, t)(codebook, indices)

    commitment_loss = jnp.sum(dmin2d) / (t * d_)
    z_q = z_q_flat.reshape(z_e.shape)
    return (z_q, indices.reshape(b, n), commitment_loss)


# speedup vs baseline: 1.2702x; 1.2702x over previous
import functools

import jax
import jax.numpy as jnp
from jax import lax
from jax.experimental import pallas as pl
from jax.experimental.pallas import tpu as pltpu
from jax.experimental.pallas import tpu_sc as plsc


_CHUNK_BOUNDS = (0, 2736, 5472, 8192)


def _dist_argmin_body(zn_ref, cn_ref, z_ref, cbt_ref,
                      idx_ref, dmin_ref,
                      v0_ref, i0_ref, v1_ref, i1_ref, v2_ref, i2_ref, *, bk):
    j = pl.program_id(1)
    nk = pl.num_programs(1)
    inf = jnp.float32(jnp.inf)
    kmax = _CHUNK_BOUNDS[-1]

    @pl.when(j == 0)
    def _():
        for vr, ir in ((v0_ref, i0_ref), (v1_ref, i1_ref), (v2_ref, i2_ref)):
            vr[...] = jnp.full(vr.shape, inf, jnp.float32)
            ir[...] = jnp.zeros(ir.shape, jnp.float32)

    mm = jnp.dot(z_ref[...], cbt_ref[...], preferred_element_type=jnp.float32)
    bm = mm.shape[0]
    kbase = (lax.broadcasted_iota(jnp.int32, (bm, 128), 1).astype(jnp.float32)
             + jnp.float32(bk) * j.astype(jnp.float32))
    fkmax = jnp.float32(kmax)
    zn = zn_ref[...]

    def update(vr, ir, lmin, larg):
        take = lmin < vr[...]
        ir[...] = jnp.where(take, larg, ir[...])
        vr[...] = jnp.where(take, lmin, vr[...])

    def pair_argmin(lo, hi, masked):
        accv = acci = None
        for c in range(bk // 128):
            sl = slice(c * 128, (c + 1) * 128)
            dcb = (zn + cn_ref[:, sl]) + mm[:, sl]
            kb = kbase + jnp.float32(c * 128)
            if masked:
                dcb = jnp.where((kb >= lo) & (kb < hi), dcb, inf)
            if accv is None:
                accv, acci = dcb, kb
            else:
                m = dcb < accv
                acci = jnp.where(m, kb, acci)
                accv = jnp.minimum(dcb, accv)
        lmin = jnp.min(accv, axis=1, keepdims=True)
        larg = jnp.min(jnp.where(accv == lmin, acci, fkmax),
                       axis=1, keepdims=True)
        return lmin, larg

    def merge_full(vr, ir):
        lmin, larg = pair_argmin(0.0, 0.0, False)
        update(vr, ir, lmin, larg)

    def merge_masked(vr, ir, lo, hi):
        lmin, larg = pair_argmin(lo, hi, True)
        update(vr, ir, lmin, larg)

    refs = ((v0_ref, i0_ref), (v1_ref, i1_ref), (v2_ref, i2_ref))
    for c in range(3):
        lo, hi = _CHUNK_BOUNDS[c], _CHUNK_BOUNDS[c + 1]
        jlo, jhi = lo // bk, (hi - 1) // bk
        pure_lo = jlo if lo % bk == 0 else jlo + 1
        pure_hi = jhi if hi % bk == 0 else jhi - 1
        if pure_lo <= pure_hi:
            @pl.when((j >= pure_lo) & (j <= pure_hi))
            def _(c=c):
                merge_full(*refs[c])
        for js in {jlo, jhi} - set(range(pure_lo, pure_hi + 1)):
            @pl.when(j == js)
            def _(c=c, lo=lo, hi=hi):
                merge_masked(*refs[c], jnp.float32(lo), jnp.float32(hi))

    @pl.when(j == nk - 1)
    def _():
        accv = v0_ref[...].astype(jnp.bfloat16).astype(jnp.float32)
        acci = i0_ref[...]
        accraw = v0_ref[...]
        for vr, ir in ((v1_ref, i1_ref), (v2_ref, i2_ref)):
            take = vr[...] < accv
            acci = jnp.where(take, ir[...], acci)
            accraw = jnp.where(take, vr[...], accraw)
            accv = jnp.where(take, vr[...], accv)
            accv = accv.astype(jnp.bfloat16).astype(jnp.float32)
        idx_ref[...] = acci.astype(jnp.int32)
        dmin_ref[...] = accraw


def _dist_argmin(flat_z, cbt, zn, cn, bm, bk):
    t, d_ = flat_z.shape
    k = cbt.shape[1]
    grid = (t // bm, k // bk)
    return pl.pallas_call(
        functools.partial(_dist_argmin_body, bk=bk),
        grid=grid,
        in_specs=[
            pl.BlockSpec((bm, 1), lambda i, j: (i, 0)),
            pl.BlockSpec((1, bk), lambda i, j: (0, j)),
            pl.BlockSpec((bm, d_), lambda i, j: (i, 0)),
            pl.BlockSpec((d_, bk), lambda i, j: (0, j)),
        ],
        out_specs=[
            pl.BlockSpec((bm, 1), lambda i, j: (i, 0)),
            pl.BlockSpec((bm, 1), lambda i, j: (i, 0)),
        ],
        out_shape=[
            jax.ShapeDtypeStruct((t, 1), jnp.int32),
            jax.ShapeDtypeStruct((t, 1), jnp.float32),
        ],
        scratch_shapes=[
            pltpu.VMEM((bm, 1), jnp.float32),
            pltpu.VMEM((bm, 1), jnp.float32),
            pltpu.VMEM((bm, 1), jnp.float32),
            pltpu.VMEM((bm, 1), jnp.float32),
            pltpu.VMEM((bm, 1), jnp.float32),
            pltpu.VMEM((bm, 1), jnp.float32),
        ],
        compiler_params=pltpu.CompilerParams(
            dimension_semantics=("parallel", "arbitrary")),
    )(zn, cn, flat_z, cbt)


def _make_sc_gather(k, d_, t):
    info = plsc.get_sparse_core_info()
    nw = info.num_cores * info.num_subcores
    b_per_w = t // nw
    ch = 128
    n_ch = b_per_w // ch
    mesh = plsc.VectorSubcoreMesh(core_axis_name="c", subcore_axis_name="s")

    @functools.partial(
        pl.kernel, mesh=mesh,
        out_type=jax.ShapeDtypeStruct((t, d_), jnp.float32),
        scratch_types=[
            pltpu.VMEM((b_per_w,), jnp.int32),
            pltpu.VMEM((ch, d_), jnp.float32),
            pltpu.VMEM((ch, d_), jnp.float32),
            pltpu.SemaphoreType.DMA,
            pltpu.SemaphoreType.DMA,
        ],
    )
    def gather_kernel(table_hbm, idx_hbm, out_hbm, idx_v, rows_a, rows_b, sem_a, sem_b):
        wid = lax.axis_index("s") * info.num_cores + lax.axis_index("c")
        base = wid * b_per_w
        pltpu.sync_copy(idx_hbm.at[pl.ds(base, b_per_w)], idx_v)
        bufs = ((rows_a, sem_a), (rows_b, sem_b))
        copies = [None, None]
        for c in range(n_ch):
            rows, sem = bufs[c % 2]
            copies[c % 2] = pltpu.async_copy(
                table_hbm.at[idx_v.at[pl.ds(c * ch, ch)]], rows, sem)
            if c >= 1:
                prows, _ = bufs[(c - 1) % 2]
                copies[(c - 1) % 2].wait()
                pltpu.sync_copy(prows, out_hbm.at[pl.ds(base + (c - 1) * ch, ch)])
        lrows, _ = bufs[(n_ch - 1) % 2]
        copies[(n_ch - 1) % 2].wait()
        pltpu.sync_copy(lrows, out_hbm.at[pl.ds(base + (n_ch - 1) * ch, ch)])

    return gather_kernel


def kernel(z_e, codebook):
    b, n, d_ = z_e.shape
    k = codebook.shape[0]
    t = b * n
    flat_z = z_e.reshape(-1, d_)
    zn = jnp.sum(flat_z ** 2, axis=1, keepdims=True)
    cn = jnp.sum(codebook ** 2, axis=1)[None, :]
    cbt = codebook.T

    idx2d, dmin2d = _dist_argmin(flat_z * jnp.float32(-2.0), cbt, zn, cn,
                                 bm=2048, bk=1024)
    indices = idx2d.reshape(t)

    z_q_flat = _make_sc_gather(k, d_, t)(codebook, indices)

    commitment_loss = jnp.sum(dmin2d) / (t * d_)
    z_q = z_q_flat.reshape(z_e.shape)
    return (z_q, indices.reshape(b, n), commitment_loss)
